# Initial kernel scaffold; baseline (speedup 1.0000x reference)
import functools

import jax
import jax.numpy as jnp
from jax import lax
from jax.experimental import pallas as pl
from jax.experimental.pallas import tpu as pltpu
from jax.experimental.pallas import tpu_sc as plsc

_NC = 2
_NS = 16
_NW = _NC * _NS
_LANES = 16

_DEG = 32
_DIN = 128
_NVREG = _DIN // _LANES

_NB = 4
_EB = _NB * _DEG


def _pad_rows(a, n_pad):
    return jnp.pad(a, ((0, n_pad - a.shape[0]),) + ((0, 0),) * (a.ndim - 1))


def _mm1_body(x_ref, wm_ref, t_ref, a_ref):
    xb = x_ref[...]
    a_ref[...] = -jnp.dot(xb, wm_ref[:_DIN, :], preferred_element_type=jnp.float32)
    t_ref[:, :_DIN] = xb
    t_ref[:, _DIN:] = -jnp.dot(xb, wm_ref[_DIN:, :], preferred_element_type=jnp.float32)


def _mm2_body(g_ref, w_ref, b_ref, o_ref):
    o_ref[...] = (
        jnp.dot(g_ref[...], w_ref[...], preferred_element_type=jnp.float32)
        + b_ref[...]
    )


def _mm1(x_pad, weights_mask, n_pad, rb):
    grid = (n_pad // rb,)
    return pl.pallas_call(
        _mm1_body,
        grid=grid,
        in_specs=[
            pl.BlockSpec((rb, _DIN), lambda i: (i, 0)),
            pl.BlockSpec((2 * _DIN, _DIN), lambda i: (0, 0)),
        ],
        out_specs=[
            pl.BlockSpec((rb, 2 * _DIN), lambda i: (i, 0)),
            pl.BlockSpec((rb, _DIN), lambda i: (i, 0)),
        ],
        out_shape=[
            jax.ShapeDtypeStruct((n_pad, 2 * _DIN), jnp.float32),
            jax.ShapeDtypeStruct((n_pad, _DIN), jnp.float32),
        ],
    )(x_pad, weights_mask)


def _mm2(agg, weight, bias2d, n_pad, rb):
    grid = (n_pad // rb,)
    return pl.pallas_call(
        _mm2_body,
        grid=grid,
        in_specs=[
            pl.BlockSpec((rb, _DIN), lambda i: (i, 0)),
            pl.BlockSpec((_DIN, _DIN), lambda i: (0, 0)),
            pl.BlockSpec((1, _DIN), lambda i: (0, 0)),
        ],
        out_specs=pl.BlockSpec((rb, _DIN), lambda i: (i, 0)),
        out_shape=jax.ShapeDtypeStruct((n_pad, _DIN), jnp.float32),
    )(agg, weight, bias2d)


def _make_pass1(n_pad, nodes_per_w):
    blocks = nodes_per_w // _NB
    mesh = plsc.VectorSubcoreMesh(core_axis_name="c", subcore_axis_name="s")

    @functools.partial(
        pl.kernel,
        out_type=jax.ShapeDtypeStruct((n_pad, _DIN), jnp.float32),
        mesh=mesh,
        scratch_types=[
            pltpu.VMEM((_EB,), jnp.int32),
            pltpu.VMEM((_EB, 2 * _DIN), jnp.float32),
            pltpu.VMEM((_NB, _DIN), jnp.float32),
            pltpu.VMEM((_NB, _DIN), jnp.float32),
            pltpu.VMEM((_NB, _DIN), jnp.float32),
            pltpu.SemaphoreType.DMA,
        ],
    )
    def pass1(t_hbm, an_hbm, x_hbm, nei_hbm, xnew_hbm, idx_v, g_v, a_v, x_v, o_v, sem):
        wid = lax.axis_index("s") * _NC + lax.axis_index("c")
        node0 = wid * nodes_per_w

        def block(bi, _):
            nb = node0 + bi * _NB
            pltpu.sync_copy(nei_hbm.at[pl.ds(nb * _DEG, _EB)], idx_v)
            pltpu.async_copy(t_hbm.at[idx_v], g_v, sem).wait()
            pltpu.sync_copy(an_hbm.at[pl.ds(nb, _NB)], a_v)
            pltpu.sync_copy(x_hbm.at[pl.ds(nb, _NB)], x_v)
            for n in range(_NB):
                accs = tuple(x_v[n, pl.ds(d * _LANES, _LANES)] for d in range(_NVREG))
                avs = tuple(a_v[n, pl.ds(d * _LANES, _LANES)] for d in range(_NVREG))

                def edge(e, accs, n=n, avs=avs):
                    r = n * _DEG + e
                    out = []
                    for d in range(_NVREG):
                        xn = g_v[r, pl.ds(d * _LANES, _LANES)]
                        bn = g_v[r, pl.ds(_DIN + d * _LANES, _LANES)]
                        u = jnp.exp(avs[d] + bn)
                        out.append(accs[d] + xn / (1.0 + u))
                    return tuple(out)

                accs = lax.fori_loop(0, _DEG, edge, accs)
                for d in range(_NVREG):
                    o_v[n, pl.ds(d * _LANES, _LANES)] = accs[d]
            pltpu.sync_copy(o_v, xnew_hbm.at[pl.ds(nb, _NB)])
            return 0

        lax.fori_loop(0, blocks, block, 0)

    return pass1


def _make_pass2(n_pad, nodes_per_w):
    blocks = nodes_per_w // _NB
    mesh = plsc.VectorSubcoreMesh(core_axis_name="c", subcore_axis_name="s")

    @functools.partial(
        pl.kernel,
        out_type=jax.ShapeDtypeStruct((n_pad, _DIN), jnp.float32),
        mesh=mesh,
        scratch_types=[
            pltpu.VMEM((_EB,), jnp.int32),
            pltpu.VMEM((_EB,), jnp.float32),
            pltpu.VMEM((_EB, _DIN), jnp.float32),
            pltpu.VMEM((_NB, _DIN), jnp.float32),
            pltpu.SemaphoreType.DMA,
        ],
    )
    def pass2(xnew_hbm, nei_hbm, adj_hbm, agg_hbm, idx_v, adj_v, g_v, o_v, sem):
        wid = lax.axis_index("s") * _NC + lax.axis_index("c")
        node0 = wid * nodes_per_w

        def block(bi, _):
            nb = node0 + bi * _NB
            pltpu.sync_copy(nei_hbm.at[pl.ds(nb * _DEG, _EB)], idx_v)
            pltpu.sync_copy(adj_hbm.at[pl.ds(nb * _DEG, _EB)], adj_v)
            pltpu.async_copy(xnew_hbm.at[idx_v], g_v, sem).wait()
            for n in range(_NB):
                accs = tuple(jnp.zeros((_LANES,), jnp.float32) for _ in range(_NVREG))

                def edge(e, accs, n=n):
                    r = n * _DEG + e
                    s = plsc.load_gather(adj_v, [jnp.full((_LANES,), r, jnp.int32)])
                    out = []
                    for d in range(_NVREG):
                        out.append(accs[d] + s * g_v[r, pl.ds(d * _LANES, _LANES)])
                    return tuple(out)

                accs = lax.fori_loop(0, _DEG, edge, accs)
                for d in range(_NVREG):
                    o_v[n, pl.ds(d * _LANES, _LANES)] = accs[d]
            pltpu.sync_copy(o_v, agg_hbm.at[pl.ds(nb, _NB)])
            return 0

        lax.fori_loop(0, blocks, block, 0)

    return pass2


def kernel(x, neighbors, adj_vals, weight, bias, weights_mask):
    n = x.shape[0]
    nodes_per_w = -(-n // (_NW * _NB)) * _NB
    n_pad = nodes_per_w * _NW

    x_pad = _pad_rows(x, n_pad)
    nei_flat = jnp.pad(neighbors.astype(jnp.int32).reshape(-1),
                       (0, (n_pad - n) * _DEG))
    adj_pad = jnp.pad(adj_vals, (0, (n_pad - n) * _DEG))

    t_tab, an = _mm1(x_pad, weights_mask, n_pad, 1024)
    x_new = _make_pass1(n_pad, nodes_per_w)(t_tab, an, x_pad, nei_flat)
    agg = _make_pass2(n_pad, nodes_per_w)(x_new, nei_flat, adj_pad)
    out = _mm2(agg, weight, bias.reshape(1, _DIN), n_pad, 1024)
    return out[:n]


# speedup vs baseline: 2.4932x; 2.4932x over previous
import functools

import jax
import jax.numpy as jnp
from jax import lax
from jax.experimental import pallas as pl
from jax.experimental.pallas import tpu as pltpu
from jax.experimental.pallas import tpu_sc as plsc

_NC = 2
_NS = 16
_NW = _NC * _NS
_LANES = 16

_DEG = 32
_DIN = 128
_NVREG = _DIN // _LANES

_SPLAT_DNUMS = lax.GatherDimensionNumbers(
    offset_dims=(), collapsed_slice_dims=(0,), start_index_map=(0,))

_NB = 4
_EB = _NB * _DEG


def _pad_rows(a, n_pad):
    return jnp.pad(a, ((0, n_pad - a.shape[0]),) + ((0, 0),) * (a.ndim - 1))


def _mm1_body(x_ref, wm_ref, t_ref, a_ref):
    xb = x_ref[...]
    a_ref[...] = -jnp.dot(xb, wm_ref[:_DIN, :], preferred_element_type=jnp.float32)
    t_ref[:, :_DIN] = xb
    t_ref[:, _DIN:] = -jnp.dot(xb, wm_ref[_DIN:, :], preferred_element_type=jnp.float32)


def _mm2_body(g_ref, w_ref, b_ref, o_ref):
    o_ref[...] = (
        jnp.dot(g_ref[...], w_ref[...], preferred_element_type=jnp.float32)
        + b_ref[...]
    )


def _mm1(x_pad, weights_mask, n_pad, rb):
    grid = (n_pad // rb,)
    return pl.pallas_call(
        _mm1_body,
        grid=grid,
        in_specs=[
            pl.BlockSpec((rb, _DIN), lambda i: (i, 0)),
            pl.BlockSpec((2 * _DIN, _DIN), lambda i: (0, 0)),
        ],
        out_specs=[
            pl.BlockSpec((rb, 2 * _DIN), lambda i: (i, 0)),
            pl.BlockSpec((rb, _DIN), lambda i: (i, 0)),
        ],
        out_shape=[
            jax.ShapeDtypeStruct((n_pad, 2 * _DIN), jnp.float32),
            jax.ShapeDtypeStruct((n_pad, _DIN), jnp.float32),
        ],
    )(x_pad, weights_mask)


def _mm2(agg, weight, bias2d, n_pad, rb):
    grid = (n_pad // rb,)
    return pl.pallas_call(
        _mm2_body,
        grid=grid,
        in_specs=[
            pl.BlockSpec((rb, _DIN), lambda i: (i, 0)),
            pl.BlockSpec((_DIN, _DIN), lambda i: (0, 0)),
            pl.BlockSpec((1, _DIN), lambda i: (0, 0)),
        ],
        out_specs=pl.BlockSpec((rb, _DIN), lambda i: (i, 0)),
        out_shape=jax.ShapeDtypeStruct((n_pad, _DIN), jnp.float32),
    )(agg, weight, bias2d)


def _make_pass1(n_pad, nodes_per_w):
    blocks = nodes_per_w // _NB
    mesh = plsc.VectorSubcoreMesh(core_axis_name="c", subcore_axis_name="s")

    @functools.partial(
        pl.kernel,
        out_type=jax.ShapeDtypeStruct((n_pad, _DIN), jnp.float32),
        mesh=mesh,
        scratch_types=[
            pltpu.VMEM((_EB,), jnp.int32),
            pltpu.VMEM((_EB, 2 * _DIN), jnp.float32),
            pltpu.VMEM((_NB, _DIN), jnp.float32),
            pltpu.VMEM((_NB, _DIN), jnp.float32),
            pltpu.VMEM((_NB, _DIN), jnp.float32),
            pltpu.SemaphoreType.DMA,
        ],
    )
    def pass1(t_hbm, an_hbm, x_hbm, nei_hbm, xnew_hbm, idx_v, g_v, a_v, x_v, o_v, sem):
        wid = lax.axis_index("s") * _NC + lax.axis_index("c")
        node0 = wid * nodes_per_w

        def block(bi, _):
            nb = node0 + bi * _NB
            pltpu.sync_copy(nei_hbm.at[pl.ds(nb * _DEG, _EB)], idx_v)
            pltpu.async_copy(t_hbm.at[idx_v], g_v, sem).wait()
            pltpu.sync_copy(an_hbm.at[pl.ds(nb, _NB)], a_v)
            pltpu.sync_copy(x_hbm.at[pl.ds(nb, _NB)], x_v)
            for n in range(_NB):
                accs = tuple(x_v[n, pl.ds(d * _LANES, _LANES)] for d in range(_NVREG))
                avs = tuple(a_v[n, pl.ds(d * _LANES, _LANES)] for d in range(_NVREG))

                def edge(e, accs, n=n, avs=avs):
                    r = n * _DEG + e
                    out = []
                    for d in range(_NVREG):
                        xn = g_v[r, pl.ds(d * _LANES, _LANES)]
                        bn = g_v[r, pl.ds(_DIN + d * _LANES, _LANES)]
                        u = jnp.exp(avs[d] + bn)
                        out.append(accs[d] + xn / (1.0 + u))
                    return tuple(out)

                accs = lax.fori_loop(0, _DEG, edge, accs)
                for d in range(_NVREG):
                    o_v[n, pl.ds(d * _LANES, _LANES)] = accs[d]
            pltpu.sync_copy(o_v, xnew_hbm.at[pl.ds(nb, _NB)])
            return 0

        lax.fori_loop(0, blocks, block, 0)

    return pass1


def _make_pass2(n_pad, nodes_per_w):
    blocks = nodes_per_w // _NB
    mesh = plsc.VectorSubcoreMesh(core_axis_name="c", subcore_axis_name="s")

    @functools.partial(
        pl.kernel,
        out_type=jax.ShapeDtypeStruct((n_pad, _DIN), jnp.float32),
        mesh=mesh,
        scratch_types=[
            pltpu.VMEM((_EB,), jnp.int32),
            pltpu.VMEM((_EB,), jnp.float32),
            pltpu.VMEM((_EB, _DIN), jnp.float32),
            pltpu.VMEM((_NB, _DIN), jnp.float32),
            pltpu.SemaphoreType.DMA,
        ],
    )
    def pass2(xnew_hbm, nei_hbm, adj_hbm, agg_hbm, idx_v, adj_v, g_v, o_v, sem):
        wid = lax.axis_index("s") * _NC + lax.axis_index("c")
        node0 = wid * nodes_per_w

        def block(bi, _):
            nb = node0 + bi * _NB
            pltpu.sync_copy(nei_hbm.at[pl.ds(nb * _DEG, _EB)], idx_v)
            pltpu.sync_copy(adj_hbm.at[pl.ds(nb * _DEG, _EB)], adj_v)
            pltpu.async_copy(xnew_hbm.at[idx_v], g_v, sem).wait()
            for n in range(_NB):
                accs = [jnp.zeros((_LANES,), jnp.float32) for _ in range(_NVREG)]
                for g in range(_DEG // _LANES):
                    av = adj_v[pl.ds(n * _DEG + g * _LANES, _LANES)]
                    for e in range(_LANES):
                        s = lax.gather(
                            av, jnp.full((_LANES, 1), e, jnp.int32),
                            _SPLAT_DNUMS, slice_sizes=(1,),
                            mode=lax.GatherScatterMode.PROMISE_IN_BOUNDS)
                        r = n * _DEG + g * _LANES + e
                        for d in range(_NVREG):
                            accs[d] = accs[d] + s * g_v[r, pl.ds(d * _LANES, _LANES)]
                for d in range(_NVREG):
                    o_v[n, pl.ds(d * _LANES, _LANES)] = accs[d]
            pltpu.sync_copy(o_v, agg_hbm.at[pl.ds(nb, _NB)])
            return 0

        lax.fori_loop(0, blocks, block, 0)

    return pass2


def kernel(x, neighbors, adj_vals, weight, bias, weights_mask):
    n = x.shape[0]
    nodes_per_w = -(-n // (_NW * _NB)) * _NB
    n_pad = nodes_per_w * _NW

    x_pad = _pad_rows(x, n_pad)
    nei_flat = jnp.pad(neighbors.astype(jnp.int32).reshape(-1),
                       (0, (n_pad - n) * _DEG))
    adj_pad = jnp.pad(adj_vals, (0, (n_pad - n) * _DEG))
    x_pad, nei_flat, adj_pad = lax.optimization_barrier((x_pad, nei_flat, adj_pad))

    t_tab, an = _mm1(x_pad, weights_mask, n_pad, n_pad)
    x_new = _make_pass1(n_pad, nodes_per_w)(t_tab, an, x_pad, nei_flat)
    agg = _make_pass2(n_pad, nodes_per_w)(x_new, nei_flat, adj_pad)
    out = _mm2(agg, weight, bias.reshape(1, _DIN), n_pad, n_pad)
    return out[:n]


# speedup vs baseline: 2.6773x; 1.0738x over previous
import functools

import jax
import jax.numpy as jnp
from jax import lax
from jax.experimental import pallas as pl
from jax.experimental.pallas import tpu as pltpu
from jax.experimental.pallas import tpu_sc as plsc

_NC = 2
_NS = 16
_NW = _NC * _NS
_LANES = 16

_DEG = 32
_DIN = 128
_NVREG = _DIN // _LANES

_SPLAT_DNUMS = lax.GatherDimensionNumbers(
    offset_dims=(), collapsed_slice_dims=(0,), start_index_map=(0,))

_NB = 4
_EB = _NB * _DEG


def _pad_rows(a, n_pad):
    return jnp.pad(a, ((0, n_pad - a.shape[0]),) + ((0, 0),) * (a.ndim - 1))


def _mm1_body(x_ref, wm_ref, t_ref, a_ref):
    xb = x_ref[...]
    a_ref[...] = -jnp.dot(xb, wm_ref[:_DIN, :], preferred_element_type=jnp.float32)
    t_ref[:, :_DIN] = xb
    t_ref[:, _DIN:] = -jnp.dot(xb, wm_ref[_DIN:, :], preferred_element_type=jnp.float32)


def _mm2_body(g_ref, w_ref, b_ref, o_ref):
    o_ref[...] = (
        jnp.dot(g_ref[...], w_ref[...], preferred_element_type=jnp.float32)
        + b_ref[...]
    )


def _mm1(x_pad, weights_mask, n_pad, rb):
    grid = (n_pad // rb,)
    return pl.pallas_call(
        _mm1_body,
        grid=grid,
        in_specs=[
            pl.BlockSpec((rb, _DIN), lambda i: (i, 0)),
            pl.BlockSpec((2 * _DIN, _DIN), lambda i: (0, 0)),
        ],
        out_specs=[
            pl.BlockSpec((rb, 2 * _DIN), lambda i: (i, 0)),
            pl.BlockSpec((rb, _DIN), lambda i: (i, 0)),
        ],
        out_shape=[
            jax.ShapeDtypeStruct((n_pad, 2 * _DIN), jnp.float32),
            jax.ShapeDtypeStruct((n_pad, _DIN), jnp.float32),
        ],
    )(x_pad, weights_mask)


def _mm2(agg, weight, bias2d, n_pad, rb):
    grid = (n_pad // rb,)
    return pl.pallas_call(
        _mm2_body,
        grid=grid,
        in_specs=[
            pl.BlockSpec((rb, _DIN), lambda i: (i, 0)),
            pl.BlockSpec((_DIN, _DIN), lambda i: (0, 0)),
            pl.BlockSpec((1, _DIN), lambda i: (0, 0)),
        ],
        out_specs=pl.BlockSpec((rb, _DIN), lambda i: (i, 0)),
        out_shape=jax.ShapeDtypeStruct((n_pad, _DIN), jnp.float32),
    )(agg, weight, bias2d)


def _make_pass1(n_pad, nodes_per_w):
    blocks = nodes_per_w // _NB
    mesh = plsc.VectorSubcoreMesh(core_axis_name="c", subcore_axis_name="s")

    @functools.partial(
        pl.kernel,
        out_type=jax.ShapeDtypeStruct((n_pad, _DIN), jnp.float32),
        mesh=mesh,
        scratch_types=[
            pltpu.VMEM((_EB,), jnp.int32),
            pltpu.VMEM((_EB, 2 * _DIN), jnp.float32),
            pltpu.VMEM((_NB, _DIN), jnp.float32),
            pltpu.VMEM((_NB, _DIN), jnp.float32),
            pltpu.VMEM((_NB, _DIN), jnp.float32),
            pltpu.SemaphoreType.DMA,
        ],
    )
    def pass1(t_hbm, an_hbm, x_hbm, nei_hbm, xnew_hbm, idx_v, g_v, a_v, x_v, o_v, sem):
        wid = lax.axis_index("s") * _NC + lax.axis_index("c")
        node0 = wid * nodes_per_w

        def block(bi, _):
            nb = node0 + bi * _NB
            pltpu.sync_copy(nei_hbm.at[pl.ds(nb * _DEG, _EB)], idx_v)
            gd = pltpu.async_copy(t_hbm.at[idx_v], g_v, sem)
            pltpu.sync_copy(an_hbm.at[pl.ds(nb, _NB)], a_v)
            pltpu.sync_copy(x_hbm.at[pl.ds(nb, _NB)], x_v)
            gd.wait()
            for n in range(_NB):
                acc0 = tuple(x_v[n, pl.ds(d * _LANES, _LANES)] for d in range(_NVREG))
                acc1 = tuple(jnp.zeros((_LANES,), jnp.float32) for _ in range(_NVREG))
                avs = tuple(a_v[n, pl.ds(d * _LANES, _LANES)] for d in range(_NVREG))

                def edge2(e2, carry, n=n, avs=avs):
                    a0, a1 = carry
                    r0 = n * _DEG + 2 * e2
                    n0, n1 = [], []
                    for d in range(_NVREG):
                        sl = pl.ds(d * _LANES, _LANES)
                        sl2 = pl.ds(_DIN + d * _LANES, _LANES)
                        u0 = jnp.exp(avs[d] + g_v[r0, sl2])
                        n0.append(a0[d] + g_v[r0, sl] / (1.0 + u0))
                        u1 = jnp.exp(avs[d] + g_v[r0 + 1, sl2])
                        n1.append(a1[d] + g_v[r0 + 1, sl] / (1.0 + u1))
                    return tuple(n0), tuple(n1)

                acc0, acc1 = lax.fori_loop(0, _DEG // 2, edge2, (acc0, acc1))
                for d in range(_NVREG):
                    o_v[n, pl.ds(d * _LANES, _LANES)] = acc0[d] + acc1[d]
            pltpu.sync_copy(o_v, xnew_hbm.at[pl.ds(nb, _NB)])
            return 0

        lax.fori_loop(0, blocks, block, 0)

    return pass1


def _make_pass2(n_pad, nodes_per_w):
    blocks = nodes_per_w // _NB
    mesh = plsc.VectorSubcoreMesh(core_axis_name="c", subcore_axis_name="s")

    @functools.partial(
        pl.kernel,
        out_type=jax.ShapeDtypeStruct((n_pad, _DIN), jnp.float32),
        mesh=mesh,
        scratch_types=[
            pltpu.VMEM((_EB,), jnp.int32),
            pltpu.VMEM((_EB,), jnp.float32),
            pltpu.VMEM((_EB, _DIN), jnp.float32),
            pltpu.VMEM((_NB, _DIN), jnp.float32),
            pltpu.SemaphoreType.DMA,
        ],
    )
    def pass2(xnew_hbm, nei_hbm, adj_hbm, agg_hbm, idx_v, adj_v, g_v, o_v, sem):
        wid = lax.axis_index("s") * _NC + lax.axis_index("c")
        node0 = wid * nodes_per_w

        def block(bi, _):
            nb = node0 + bi * _NB
            pltpu.sync_copy(nei_hbm.at[pl.ds(nb * _DEG, _EB)], idx_v)
            pltpu.sync_copy(adj_hbm.at[pl.ds(nb * _DEG, _EB)], adj_v)
            pltpu.async_copy(xnew_hbm.at[idx_v], g_v, sem).wait()
            for n in range(_NB):
                accs = [jnp.zeros((_LANES,), jnp.float32) for _ in range(_NVREG)]
                for g in range(_DEG // _LANES):
                    av = adj_v[pl.ds(n * _DEG + g * _LANES, _LANES)]
                    for e in range(_LANES):
                        s = lax.gather(
                            av, jnp.full((_LANES, 1), e, jnp.int32),
                            _SPLAT_DNUMS, slice_sizes=(1,),
                            mode=lax.GatherScatterMode.PROMISE_IN_BOUNDS)
                        r = n * _DEG + g * _LANES + e
                        for d in range(_NVREG):
                            accs[d] = accs[d] + s * g_v[r, pl.ds(d * _LANES, _LANES)]
                for d in range(_NVREG):
                    o_v[n, pl.ds(d * _LANES, _LANES)] = accs[d]
            pltpu.sync_copy(o_v, agg_hbm.at[pl.ds(nb, _NB)])
            return 0

        lax.fori_loop(0, blocks, block, 0)

    return pass2


def kernel(x, neighbors, adj_vals, weight, bias, weights_mask):
    n = x.shape[0]
    nodes_per_w = -(-n // (_NW * _NB)) * _NB
    n_pad = nodes_per_w * _NW

    x_pad = _pad_rows(x, n_pad)
    nei_flat = jnp.pad(neighbors.astype(jnp.int32).reshape(-1),
                       (0, (n_pad - n) * _DEG))
    adj_pad = jnp.pad(adj_vals, (0, (n_pad - n) * _DEG))
    x_pad, nei_flat, adj_pad = lax.optimization_barrier((x_pad, nei_flat, adj_pad))

    t_tab, an = _mm1(x_pad, weights_mask, n_pad, n_pad)
    x_new = _make_pass1(n_pad, nodes_per_w)(t_tab, an, x_pad, nei_flat)
    agg = _make_pass2(n_pad, nodes_per_w)(x_new, nei_flat, adj_pad)
    out = _mm2(agg, weight, bias.reshape(1, _DIN), n_pad, n_pad)
    return out[:n]


# speedup vs baseline: 2.9628x; 1.1066x over previous
import functools

import jax
import jax.numpy as jnp
from jax import lax
from jax.experimental import pallas as pl
from jax.experimental.pallas import tpu as pltpu
from jax.experimental.pallas import tpu_sc as plsc

_NC = 2
_NS = 16
_NW = _NC * _NS
_LANES = 16

_DEG = 32
_DIN = 128
_NVREG = _DIN // _LANES

_SPLAT_DNUMS = lax.GatherDimensionNumbers(
    offset_dims=(), collapsed_slice_dims=(0,), start_index_map=(0,))

_NB = 4
_EB = _NB * _DEG


def _pad_rows(a, n_pad):
    return jnp.pad(a, ((0, n_pad - a.shape[0]),) + ((0, 0),) * (a.ndim - 1))


def _mm1_body(x_ref, wm_ref, t_ref, ax_ref):
    xb = x_ref[...]
    ax_ref[:, :_DIN] = -jnp.dot(xb, wm_ref[:_DIN, :], preferred_element_type=jnp.float32)
    ax_ref[:, _DIN:] = xb
    t_ref[:, :_DIN] = xb
    t_ref[:, _DIN:] = -jnp.dot(xb, wm_ref[_DIN:, :], preferred_element_type=jnp.float32)


def _mm2_body(g_ref, w_ref, b_ref, o_ref):
    o_ref[...] = (
        jnp.dot(g_ref[...], w_ref[...], preferred_element_type=jnp.float32)
        + b_ref[...]
    )


def _mm1(x_pad, weights_mask, n_pad, rb):
    grid = (n_pad // rb,)
    return pl.pallas_call(
        _mm1_body,
        grid=grid,
        in_specs=[
            pl.BlockSpec((rb, _DIN), lambda i: (i, 0)),
            pl.BlockSpec((2 * _DIN, _DIN), lambda i: (0, 0)),
        ],
        out_specs=[
            pl.BlockSpec((rb, 2 * _DIN), lambda i: (i, 0)),
            pl.BlockSpec((rb, 2 * _DIN), lambda i: (i, 0)),
        ],
        out_shape=[
            jax.ShapeDtypeStruct((n_pad, 2 * _DIN), jnp.float32),
            jax.ShapeDtypeStruct((n_pad, 2 * _DIN), jnp.float32),
        ],
    )(x_pad, weights_mask)


def _mm2(agg, weight, bias2d, n_pad, rb):
    grid = (n_pad // rb,)
    return pl.pallas_call(
        _mm2_body,
        grid=grid,
        in_specs=[
            pl.BlockSpec((rb, _DIN), lambda i: (i, 0)),
            pl.BlockSpec((_DIN, _DIN), lambda i: (0, 0)),
            pl.BlockSpec((1, _DIN), lambda i: (0, 0)),
        ],
        out_specs=pl.BlockSpec((rb, _DIN), lambda i: (i, 0)),
        out_shape=jax.ShapeDtypeStruct((n_pad, _DIN), jnp.float32),
    )(agg, weight, bias2d)


def _make_pass1(n_pad, nodes_per_w):
    blocks = nodes_per_w // _NB
    mesh = plsc.VectorSubcoreMesh(core_axis_name="c", subcore_axis_name="s")

    @functools.partial(
        pl.kernel,
        out_type=jax.ShapeDtypeStruct((n_pad, _DIN), jnp.float32),
        mesh=mesh,
        scratch_types=[
            pltpu.VMEM((_EB,), jnp.int32),
            pltpu.VMEM((_EB, 2 * _DIN), jnp.float32),
            pltpu.VMEM((_NB, 2 * _DIN), jnp.float32),
            pltpu.VMEM((_NB, _DIN), jnp.float32),
            pltpu.SemaphoreType.DMA,
        ],
    )
    def pass1(t_hbm, ax_hbm, nei_hbm, xnew_hbm, idx_v, g_v, ax_v, o_v, sem):
        wid = lax.axis_index("s") * _NC + lax.axis_index("c")
        node0 = wid * nodes_per_w

        def block(bi, _):
            nb = node0 + bi * _NB
            pltpu.sync_copy(nei_hbm.at[pl.ds(nb * _DEG, _EB)], idx_v)
            gd = pltpu.async_copy(t_hbm.at[idx_v], g_v, sem)
            pltpu.sync_copy(ax_hbm.at[pl.ds(nb, _NB)], ax_v)
            gd.wait()
            for n in range(_NB):
                acc0 = tuple(ax_v[n, pl.ds(_DIN + d * _LANES, _LANES)]
                             for d in range(_NVREG))
                acc1 = tuple(jnp.zeros((_LANES,), jnp.float32) for _ in range(_NVREG))
                avs = tuple(ax_v[n, pl.ds(d * _LANES, _LANES)] for d in range(_NVREG))

                def edge2(e2, carry, n=n, avs=avs):
                    a0, a1 = carry
                    r0 = n * _DEG + 2 * e2
                    n0, n1 = [], []
                    for d in range(_NVREG):
                        sl = pl.ds(d * _LANES, _LANES)
                        sl2 = pl.ds(_DIN + d * _LANES, _LANES)
                        u0 = jnp.exp(avs[d] + g_v[r0, sl2])
                        n0.append(a0[d] + g_v[r0, sl] / (1.0 + u0))
                        u1 = jnp.exp(avs[d] + g_v[r0 + 1, sl2])
                        n1.append(a1[d] + g_v[r0 + 1, sl] / (1.0 + u1))
                    return tuple(n0), tuple(n1)

                acc0, acc1 = lax.fori_loop(0, _DEG // 2, edge2, (acc0, acc1))
                for d in range(_NVREG):
                    o_v[n, pl.ds(d * _LANES, _LANES)] = acc0[d] + acc1[d]
            pltpu.sync_copy(o_v, xnew_hbm.at[pl.ds(nb, _NB)])
            return 0

        lax.fori_loop(0, blocks, block, 0)

    return pass1


def _make_pass2(n_pad, nodes_per_w):
    blocks = nodes_per_w // _NB
    mesh = plsc.VectorSubcoreMesh(core_axis_name="c", subcore_axis_name="s")

    @functools.partial(
        pl.kernel,
        out_type=jax.ShapeDtypeStruct((n_pad, _DIN), jnp.float32),
        mesh=mesh,
        scratch_types=[
            pltpu.VMEM((_EB,), jnp.int32),
            pltpu.VMEM((_EB,), jnp.float32),
            pltpu.VMEM((_EB, _DIN), jnp.float32),
            pltpu.VMEM((_NB, _DIN), jnp.float32),
            pltpu.SemaphoreType.DMA,
        ],
    )
    def pass2(xnew_hbm, nei_hbm, adj_hbm, agg_hbm, idx_v, adj_v, g_v, o_v, sem):
        wid = lax.axis_index("s") * _NC + lax.axis_index("c")
        node0 = wid * nodes_per_w

        def block(bi, _):
            nb = node0 + bi * _NB
            pltpu.sync_copy(nei_hbm.at[pl.ds(nb * _DEG, _EB)], idx_v)
            gd = pltpu.async_copy(xnew_hbm.at[idx_v], g_v, sem)
            pltpu.sync_copy(adj_hbm.at[pl.ds(nb * _DEG, _EB)], adj_v)
            gd.wait()
            for n in range(_NB):
                accs = [jnp.zeros((_LANES,), jnp.float32) for _ in range(_NVREG)]
                for g in range(_DEG // _LANES):
                    av = adj_v[pl.ds(n * _DEG + g * _LANES, _LANES)]
                    for e in range(_LANES):
                        s = lax.gather(
                            av, jnp.full((_LANES, 1), e, jnp.int32),
                            _SPLAT_DNUMS, slice_sizes=(1,),
                            mode=lax.GatherScatterMode.PROMISE_IN_BOUNDS)
                        r = n * _DEG + g * _LANES + e
                        for d in range(_NVREG):
                            accs[d] = accs[d] + s * g_v[r, pl.ds(d * _LANES, _LANES)]
                for d in range(_NVREG):
                    o_v[n, pl.ds(d * _LANES, _LANES)] = accs[d]
            pltpu.sync_copy(o_v, agg_hbm.at[pl.ds(nb, _NB)])
            return 0

        lax.fori_loop(0, blocks, block, 0)

    return pass2


def kernel(x, neighbors, adj_vals, weight, bias, weights_mask):
    n = x.shape[0]
    nodes_per_w = -(-n // (_NW * _NB)) * _NB
    n_pad = nodes_per_w * _NW

    x_pad = _pad_rows(x, n_pad)
    nei_flat = jnp.pad(neighbors.astype(jnp.int32).reshape(-1),
                       (0, (n_pad - n) * _DEG))
    adj_pad = jnp.pad(adj_vals, (0, (n_pad - n) * _DEG))
    x_pad, nei_flat, adj_pad = lax.optimization_barrier((x_pad, nei_flat, adj_pad))

    t_tab, ax = _mm1(x_pad, weights_mask, n_pad, n_pad)
    x_new = _make_pass1(n_pad, nodes_per_w)(t_tab, ax, nei_flat)
    agg = _make_pass2(n_pad, nodes_per_w)(x_new, nei_flat, adj_pad)
    out = _mm2(agg, weight, bias.reshape(1, _DIN), n_pad, n_pad)
    return out[:n]
